# baseline (device time: 69130 ns/iter reference)
import jax
import jax.numpy as jnp
from jax import lax
from jax.experimental import pallas as pl
from jax.experimental.pallas import tpu as pltpu

N_DEV = 16
SQ = 256
SKV = 4096
D_MODEL = 1024
HQ_LOCAL = 8
DH = 128
HD_LOCAL = HQ_LOCAL * DH
QB = 4
KSEL = SKV // QB
SCALE = 0.08838834764831843
ROWS = SQ // N_DEV
NBUF = 4
STEPS = [(h, r) for h in range(HQ_LOCAL) for r in range(QB)]

_CompilerParams = getattr(pltpu, "CompilerParams", None) or getattr(
    pltpu, "TPUCompilerParams"
)


def _body(x_ref, wq_hbm, k_hbm, v_hbm, wo_hbm, out_ref,
          wq_ref, wo_ref, k_buf, v_buf, ctx_ref, rs_buf, ag_src, ag_buf,
          w_sems, k_sems, v_sems,
          rs_send_sems, rs_recv_sems, ag_send_sems, ag_recv_sems):
    my = lax.axis_index("i")

    wq_dma = pltpu.make_async_copy(
        wq_hbm.at[:, pl.ds(my * HD_LOCAL, HD_LOCAL)], wq_ref, w_sems.at[0])
    wo_dma = pltpu.make_async_copy(
        wo_hbm.at[pl.ds(my * HD_LOCAL, HD_LOCAL), :], wo_ref, w_sems.at[1])
    wq_dma.start()
    wo_dma.start()

    def kv_dma(idx):
        h, r = STEPS[idx]
        slot = idx % NBUF
        return (
            pltpu.make_async_copy(k_hbm.at[h, :, r, :, :], k_buf.at[slot],
                                  k_sems.at[slot]),
            pltpu.make_async_copy(v_hbm.at[h, :, r, :, :], v_buf.at[slot],
                                  v_sems.at[slot]),
        )

    for idx in range(NBUF - 1):
        kd, vd = kv_dma(idx)
        kd.start()
        vd.start()

    barrier_sem = pltpu.get_barrier_semaphore()
    for p in range(N_DEV):
        pl.semaphore_signal(barrier_sem, inc=1, device_id=(p,),
                            device_id_type=pl.DeviceIdType.MESH)
    pl.semaphore_wait(barrier_sem, N_DEV)

    wq_dma.wait()
    x_bf = x_ref[...].astype(jnp.bfloat16)
    q = jnp.dot(x_bf, wq_ref[...].astype(jnp.bfloat16),
                preferred_element_type=jnp.float32)
    q = q.astype(jnp.bfloat16)

    for idx, (h, r) in enumerate(STEPS):
        if idx + NBUF - 1 < len(STEPS):
            kd, vd = kv_dma(idx + NBUF - 1)
            kd.start()
            vd.start()
        kw, vw = kv_dma(idx)
        kw.wait()
        vw.wait()
        slot = idx % NBUF
        k_mat = k_buf[slot].reshape(KSEL, DH)
        v_mat = v_buf[slot].reshape(KSEL, DH)
        q_blk = q[r * 64:(r + 1) * 64, h * DH:(h + 1) * DH]
        s = lax.dot_general(q_blk, k_mat, (((1,), (1,)), ((), ())),
                            preferred_element_type=jnp.float32)
        s = s * SCALE
        m = jnp.max(s, axis=-1, keepdims=True)
        e = jnp.exp(s - m)
        w = (e / jnp.sum(e, axis=-1, keepdims=True)).astype(jnp.bfloat16)
        ctx_blk = jnp.dot(w, v_mat, preferred_element_type=jnp.float32)
        ctx_ref[r * 64:(r + 1) * 64, h * DH:(h + 1) * DH] = (
            ctx_blk.astype(jnp.bfloat16))

    wo_dma.wait()
    partial = jnp.dot(ctx_ref[...], wo_ref[...].astype(jnp.bfloat16),
                      preferred_element_type=jnp.float32)
    partial_bf = partial.astype(jnp.bfloat16)

    for j in range(N_DEV):
        rs_buf[j] = partial_bf[j * ROWS:(j + 1) * ROWS, :]

    rs_sends = []
    for j in range(N_DEV):
        rdma = pltpu.make_async_remote_copy(
            src_ref=rs_buf.at[j],
            dst_ref=ag_buf.at[my],
            send_sem=rs_send_sems.at[j],
            recv_sem=rs_recv_sems.at[my],
            device_id=(j,),
            device_id_type=pl.DeviceIdType.MESH,
        )
        rdma.start()
        rs_sends.append(rdma)

    for s in range(N_DEV):
        pltpu.make_async_remote_copy(
            src_ref=ag_buf.at[s], dst_ref=ag_buf.at[s],
            send_sem=rs_send_sems.at[s], recv_sem=rs_recv_sems.at[s],
            device_id=(0,), device_id_type=pl.DeviceIdType.MESH,
        ).wait_recv()
    red = ag_buf[0].astype(jnp.float32)
    for s in range(1, N_DEV):
        red = red + ag_buf[s].astype(jnp.float32)
    ag_src[...] = red.astype(jnp.bfloat16)

    ag_sends = []
    for j in range(N_DEV):
        rdma = pltpu.make_async_remote_copy(
            src_ref=ag_src,
            dst_ref=rs_buf.at[my],
            send_sem=ag_send_sems.at[j],
            recv_sem=ag_recv_sems.at[my],
            device_id=(j,),
            device_id_type=pl.DeviceIdType.MESH,
        )
        rdma.start()
        ag_sends.append(rdma)

    for j in range(N_DEV):
        pltpu.make_async_remote_copy(
            src_ref=rs_buf.at[j], dst_ref=rs_buf.at[j],
            send_sem=ag_send_sems.at[j], recv_sem=ag_recv_sems.at[j],
            device_id=(0,), device_id_type=pl.DeviceIdType.MESH,
        ).wait_recv()
        out_ref[j * ROWS:(j + 1) * ROWS, :] = rs_buf[j].astype(jnp.float32)

    for rdma in rs_sends + ag_sends:
        rdma.wait_send()


def kernel(x, Wq, K_ext, V_ext, Wo):
    x2 = x[0]
    k5 = jnp.transpose(K_ext[0], (1, 0, 2)).astype(jnp.bfloat16).reshape(
        HQ_LOCAL, 16, QB, 64, DH)
    v5 = jnp.transpose(V_ext[0], (1, 0, 2)).astype(jnp.bfloat16).reshape(
        HQ_LOCAL, 16, QB, 64, DH)

    out2 = pl.pallas_call(
        _body,
        out_shape=jax.ShapeDtypeStruct((SQ, D_MODEL), jnp.float32),
        in_specs=[
            pl.BlockSpec(memory_space=pltpu.VMEM),
            pl.BlockSpec(memory_space=pltpu.MemorySpace.HBM),
            pl.BlockSpec(memory_space=pltpu.MemorySpace.HBM),
            pl.BlockSpec(memory_space=pltpu.MemorySpace.HBM),
            pl.BlockSpec(memory_space=pltpu.MemorySpace.HBM),
        ],
        out_specs=pl.BlockSpec(memory_space=pltpu.VMEM),
        scratch_shapes=[
            pltpu.VMEM((D_MODEL, HD_LOCAL), jnp.float32),
            pltpu.VMEM((HD_LOCAL, D_MODEL), jnp.float32),
            pltpu.VMEM((NBUF, 16, 64, DH), jnp.bfloat16),
            pltpu.VMEM((NBUF, 16, 64, DH), jnp.bfloat16),
            pltpu.VMEM((SQ, HD_LOCAL), jnp.bfloat16),
            pltpu.VMEM((N_DEV, ROWS, D_MODEL), jnp.bfloat16),
            pltpu.VMEM((ROWS, D_MODEL), jnp.bfloat16),
            pltpu.VMEM((N_DEV, ROWS, D_MODEL), jnp.bfloat16),
            pltpu.SemaphoreType.DMA((2,)),
            pltpu.SemaphoreType.DMA((NBUF,)),
            pltpu.SemaphoreType.DMA((NBUF,)),
            pltpu.SemaphoreType.DMA((N_DEV,)),
            pltpu.SemaphoreType.DMA((N_DEV,)),
            pltpu.SemaphoreType.DMA((N_DEV,)),
            pltpu.SemaphoreType.DMA((N_DEV,)),
        ],
        compiler_params=_CompilerParams(collective_id=0),
    )(x2, Wq, k5, v5, Wo)
    return out2[None]


# device time: 58738 ns/iter; 1.1769x vs baseline; 1.1769x over previous
import jax
import jax.numpy as jnp
from jax import lax
from jax.experimental import pallas as pl
from jax.experimental.pallas import tpu as pltpu

N_DEV = 16
SQ = 256
SKV = 4096
D_MODEL = 1024
HQ_LOCAL = 8
DH = 128
HD_LOCAL = HQ_LOCAL * DH
SCALE = 0.08838834764831843
ROWS = SQ // N_DEV

_CompilerParams = getattr(pltpu, "CompilerParams", None) or getattr(
    pltpu, "TPUCompilerParams"
)


def _body(x_ref, wq_hbm, k_hbm, v_hbm, wo_hbm, out_ref,
          wq_ref, wo_ref, k_buf, v_buf, ctx_ref, rs_buf, ag_src, ag_buf,
          w_sems, k_sems, v_sems,
          rs_send_sems, rs_recv_sems, ag_send_sems, ag_recv_sems):
    my = lax.axis_index("i")

    wq_dma = pltpu.make_async_copy(
        wq_hbm.at[:, pl.ds(my * HD_LOCAL, HD_LOCAL)], wq_ref, w_sems.at[0])
    wo_dma = pltpu.make_async_copy(
        wo_hbm.at[pl.ds(my * HD_LOCAL, HD_LOCAL), :], wo_ref, w_sems.at[1])
    wq_dma.start()
    wo_dma.start()

    NBUF = 4
    def kv_dma(h):
        slot = h % NBUF
        return (
            pltpu.make_async_copy(k_hbm.at[:, h, :], k_buf.at[slot],
                                  k_sems.at[slot]),
            pltpu.make_async_copy(v_hbm.at[:, h, :], v_buf.at[slot],
                                  v_sems.at[slot]),
        )

    for hh in range(NBUF - 1):
        kd0, vd0 = kv_dma(hh)
        kd0.start()
        vd0.start()

    barrier_sem = pltpu.get_barrier_semaphore()
    for p in range(N_DEV):
        pl.semaphore_signal(barrier_sem, inc=1, device_id=(p,),
                            device_id_type=pl.DeviceIdType.MESH)
    pl.semaphore_wait(barrier_sem, N_DEV)

    wq_dma.wait()
    x_bf = x_ref[...].astype(jnp.bfloat16)
    q = jnp.dot(x_bf, wq_ref[...].astype(jnp.bfloat16),
                preferred_element_type=jnp.float32)
    q = q.astype(jnp.bfloat16)

    qb = lax.broadcasted_iota(jnp.int32, (SQ, SKV), 0) // 64
    kb = lax.broadcasted_iota(jnp.int32, (SQ, SKV), 1) // 64
    mask = (kb % 4) == (qb % 4)

    kv_waits = [kv_dma(h) for h in range(HQ_LOCAL)]
    for h in range(HQ_LOCAL):
        if h + NBUF - 1 < HQ_LOCAL:
            kd, vd = kv_dma(h + NBUF - 1)
            kd.start()
            vd.start()
        kw, vw = kv_waits[h]
        kw.wait()
        vw.wait()
        slot = h % NBUF
        q_h = q[:, h * DH:(h + 1) * DH]
        k_h = k_buf[slot].astype(jnp.bfloat16)
        v_h = v_buf[slot].astype(jnp.bfloat16)
        s = lax.dot_general(q_h, k_h, (((1,), (1,)), ((), ())),
                            preferred_element_type=jnp.float32)
        s = jnp.where(mask, s * SCALE, -1e9)
        m = jnp.max(s, axis=-1, keepdims=True)
        e = jnp.exp(s - m)
        w = (e / jnp.sum(e, axis=-1, keepdims=True)).astype(jnp.bfloat16)
        ctx_h = jnp.dot(w, v_h, preferred_element_type=jnp.float32)
        ctx_ref[:, h * DH:(h + 1) * DH] = ctx_h.astype(jnp.bfloat16)

    wo_dma.wait()
    partial = jnp.dot(ctx_ref[...], wo_ref[...].astype(jnp.bfloat16),
                      preferred_element_type=jnp.float32)
    partial_bf = partial.astype(jnp.bfloat16)

    for j in range(N_DEV):
        rs_buf[j] = partial_bf[j * ROWS:(j + 1) * ROWS, :]

    rs_sends = []
    for j in range(N_DEV):
        rdma = pltpu.make_async_remote_copy(
            src_ref=rs_buf.at[j],
            dst_ref=ag_buf.at[my],
            send_sem=rs_send_sems.at[j],
            recv_sem=rs_recv_sems.at[my],
            device_id=(j,),
            device_id_type=pl.DeviceIdType.MESH,
        )
        rdma.start()
        rs_sends.append(rdma)

    for s in range(N_DEV):
        pltpu.make_async_remote_copy(
            src_ref=ag_buf.at[s], dst_ref=ag_buf.at[s],
            send_sem=rs_send_sems.at[s], recv_sem=rs_recv_sems.at[s],
            device_id=(0,), device_id_type=pl.DeviceIdType.MESH,
        ).wait_recv()
    red = ag_buf[0].astype(jnp.float32)
    for s in range(1, N_DEV):
        red = red + ag_buf[s].astype(jnp.float32)
    ag_src[...] = red.astype(jnp.bfloat16)

    ag_sends = []
    for j in range(N_DEV):
        rdma = pltpu.make_async_remote_copy(
            src_ref=ag_src,
            dst_ref=rs_buf.at[my],
            send_sem=ag_send_sems.at[j],
            recv_sem=ag_recv_sems.at[my],
            device_id=(j,),
            device_id_type=pl.DeviceIdType.MESH,
        )
        rdma.start()
        ag_sends.append(rdma)

    for j in range(N_DEV):
        pltpu.make_async_remote_copy(
            src_ref=rs_buf.at[j], dst_ref=rs_buf.at[j],
            send_sem=ag_send_sems.at[j], recv_sem=ag_recv_sems.at[j],
            device_id=(0,), device_id_type=pl.DeviceIdType.MESH,
        ).wait_recv()
        out_ref[j * ROWS:(j + 1) * ROWS, :] = rs_buf[j].astype(jnp.float32)

    for rdma in rs_sends + ag_sends:
        rdma.wait_send()


def kernel(x, Wq, K_ext, V_ext, Wo):
    x2 = x[0]
    k2 = K_ext[0]
    v2 = V_ext[0]

    out2 = pl.pallas_call(
        _body,
        out_shape=jax.ShapeDtypeStruct((SQ, D_MODEL), jnp.float32),
        in_specs=[
            pl.BlockSpec(memory_space=pltpu.VMEM),
            pl.BlockSpec(memory_space=pltpu.MemorySpace.HBM),
            pl.BlockSpec(memory_space=pltpu.MemorySpace.HBM),
            pl.BlockSpec(memory_space=pltpu.MemorySpace.HBM),
            pl.BlockSpec(memory_space=pltpu.MemorySpace.HBM),
        ],
        out_specs=pl.BlockSpec(memory_space=pltpu.VMEM),
        scratch_shapes=[
            pltpu.VMEM((D_MODEL, HD_LOCAL), jnp.float32),
            pltpu.VMEM((HD_LOCAL, D_MODEL), jnp.float32),
            pltpu.VMEM((4, SKV, DH), jnp.float32),
            pltpu.VMEM((4, SKV, DH), jnp.float32),
            pltpu.VMEM((SQ, HD_LOCAL), jnp.bfloat16),
            pltpu.VMEM((N_DEV, ROWS, D_MODEL), jnp.bfloat16),
            pltpu.VMEM((ROWS, D_MODEL), jnp.bfloat16),
            pltpu.VMEM((N_DEV, ROWS, D_MODEL), jnp.bfloat16),
            pltpu.SemaphoreType.DMA((2,)),
            pltpu.SemaphoreType.DMA((4,)),
            pltpu.SemaphoreType.DMA((4,)),
            pltpu.SemaphoreType.DMA((N_DEV,)),
            pltpu.SemaphoreType.DMA((N_DEV,)),
            pltpu.SemaphoreType.DMA((N_DEV,)),
            pltpu.SemaphoreType.DMA((N_DEV,)),
        ],
        compiler_params=_CompilerParams(collective_id=0),
    )(x2, Wq, k2, v2, Wo)
    return out2[None]
